# baseline (speedup 1.0000x reference)
import functools

import jax
import jax.numpy as jnp
from jax import lax
from jax.experimental import pallas as pl
from jax.experimental.pallas import tpu as pltpu
from jax.experimental.pallas import tpu_sc as plsc

_LANES = 16
_WINDOW = 1664
_PERIOD = 208


def kernel(categorical_inputs, tables):
    B, T = categorical_inputs.shape
    _, vocab, D = tables.shape
    R = B * T

    tab_flat = tables.reshape(T * vocab, D)
    idx_flat = categorical_inputs.reshape(1, R)

    mesh = plsc.VectorSubcoreMesh(core_axis_name="core", subcore_axis_name="subcore")

    @functools.partial(
        pl.kernel,
        out_type=jax.ShapeDtypeStruct((R, D), tables.dtype),
        mesh=mesh,
        compiler_params=pltpu.CompilerParams(use_tc_tiling_on_sc=False),
        scratch_types=[
            pltpu.VMEM((_PERIOD,), jnp.int32),
            pltpu.VMEM((_WINDOW,), jnp.int32),
        ],
    )
    def run(tab_hbm, idx_hbm, out_hbm, off_v, adj_v):
        for j in range(_PERIOD // _LANES):
            lane = lax.iota(jnp.int32, _LANES) + (j * _LANES)
            off_v[pl.ds(j * _LANES, _LANES)] = (lane % T) * vocab

        def body(i_vmem, o_vmem):
            for k in range(0, _WINDOW, _LANES):
                adj_v[pl.ds(k, _LANES)] = (
                    i_vmem[0, pl.ds(k, _LANES)]
                    + off_v[pl.ds(k % _PERIOD, _LANES)]
                )
            pltpu.sync_copy(tab_hbm.at[adj_v], o_vmem)

        pltpu.emit_pipeline(
            body,
            grid=(R // _WINDOW,),
            in_specs=[pl.BlockSpec((1, _WINDOW), index_map=lambda i: (0, i))],
            out_specs=[pl.BlockSpec((_WINDOW, D), index_map=lambda i: (i, 0))],
            core_axis_name=("core", "subcore"),
            dimension_semantics=(pltpu.PARALLEL,),
        )(idx_hbm, out_hbm)

    return run(tab_flat, idx_flat).reshape(B, T, D)


# speedup vs baseline: 1.0003x; 1.0003x over previous
import functools

import jax
import jax.numpy as jnp
from jax import lax
from jax.experimental import pallas as pl
from jax.experimental.pallas import tpu as pltpu
from jax.experimental.pallas import tpu_sc as plsc

_LANES = 16
_NWORKERS = 32


def kernel(categorical_inputs, tables):
    B, T = categorical_inputs.shape
    _, vocab, D = tables.shape
    BB = B // _NWORKERS

    mesh = plsc.VectorSubcoreMesh(core_axis_name="core", subcore_axis_name="subcore")

    @functools.partial(
        pl.kernel,
        out_type=jax.ShapeDtypeStruct((B, T, D), tables.dtype),
        mesh=mesh,
        compiler_params=pltpu.CompilerParams(
            use_tc_tiling_on_sc=False, needs_layout_passes=False
        ),
        scratch_types=[
            pltpu.VMEM((BB, T), jnp.int32),
            pltpu.VMEM((2, BB), jnp.int32),
            pltpu.VMEM((3, BB, D), jnp.float32),
            pltpu.SemaphoreType.DMA,
            pltpu.SemaphoreType.DMA,
        ],
    )
    def run(tab_hbm, cat_hbm, out_hbm, catv, idxv, rowv, gsem, osem):
        wid = lax.axis_index("subcore") * 2 + lax.axis_index("core")
        b0 = wid * BB

        pltpu.sync_copy(cat_hbm.at[pl.ds(b0, BB), :], catv)

        def build(t):
            col = jnp.full((_LANES,), t, jnp.int32)
            for j in range(BB // _LANES):
                rows = lax.iota(jnp.int32, _LANES) + (j * _LANES)
                idxv[t % 2, pl.ds(j * _LANES, _LANES)] = plsc.load_gather(
                    catv, [rows, col]
                )

        def gather(t):
            return pltpu.async_copy(
                tab_hbm.at[t].at[idxv.at[t % 2]], rowv.at[t % 3], gsem
            )

        def write(t):
            return pltpu.async_copy(
                rowv.at[t % 3], out_hbm.at[pl.ds(b0, BB), t, :], osem
            )

        build(0)
        g = {0: gather(0)}
        build(1)
        g[1] = gather(1)
        w = {}
        for t in range(T):
            g[t].wait()
            w[t] = write(t)
            if t + 2 < T:
                if t - 1 >= 0:
                    w[t - 1].wait()
                build(t + 2)
                g[t + 2] = gather(t + 2)
        w[T - 3].wait()
        w[T - 2].wait()
        w[T - 1].wait()

    return run(tables, categorical_inputs)


# speedup vs baseline: 3.6698x; 3.6687x over previous
import functools

import jax
import jax.numpy as jnp
from jax import lax
from jax.experimental import pallas as pl
from jax.experimental.pallas import tpu as pltpu
from jax.experimental.pallas import tpu_sc as plsc

_LANES = 16
_NWORKERS = 32
_CHUNK = 4096


def kernel(categorical_inputs, tables):
    B, T = categorical_inputs.shape
    _, V, D = tables.shape

    tab_t = jnp.transpose(tables, (0, 2, 1))
    cat_t = categorical_inputs.T

    n_pairs = T * D // _NWORKERS

    mesh = plsc.VectorSubcoreMesh(core_axis_name="core", subcore_axis_name="subcore")

    @functools.partial(
        pl.kernel,
        out_type=jax.ShapeDtypeStruct((T, D, B), tables.dtype),
        mesh=mesh,
        compiler_params=pltpu.CompilerParams(needs_layout_passes=False),
        scratch_types=[
            pltpu.VMEM((V,), jnp.float32),
            pltpu.VMEM((B,), jnp.int32),
            pltpu.VMEM((2, _CHUNK), jnp.float32),
            pltpu.SemaphoreType.DMA,
        ],
    )
    def run(tab_hbm, cat_hbm, out_hbm, vrow, idxv, outb, osem):
        wid = lax.axis_index("subcore") * 2 + lax.axis_index("core")
        p0 = wid * n_pairs

        @pl.loop(0, n_pairs)
        def pair(i):
            p = p0 + i
            t = lax.shift_right_logical(p, 5)
            d = jnp.bitwise_and(p, 31)

            @pl.when(jnp.logical_or(d == 0, i == 0))
            def _():
                pltpu.sync_copy(cat_hbm.at[t, :], idxv)

            pltpu.sync_copy(tab_hbm.at[t, d, :], vrow)

            writes = []
            for c in range(B // _CHUNK):
                s = c % 2
                if c >= 2:
                    writes[c - 2].wait()

                @pl.loop(0, _CHUNK // _LANES, step=8)
                def gath(j):
                    for u in range(8):
                        off = (j + u) * _LANES
                        iv = idxv[pl.ds(c * _CHUNK + off, _LANES)]
                        outb[s, pl.ds(off, _LANES)] = plsc.load_gather(vrow, [iv])

                writes.append(
                    pltpu.async_copy(
                        outb.at[s], out_hbm.at[t, d, pl.ds(c * _CHUNK, _CHUNK)], osem
                    )
                )
            writes[-2].wait()
            writes[-1].wait()

    out_t = run(tab_t, cat_t)
    return jnp.transpose(out_t, (2, 0, 1))


# speedup vs baseline: 6.9779x; 1.9015x over previous
import functools

import jax
import jax.numpy as jnp
from jax import lax
from jax.experimental import pallas as pl
from jax.experimental.pallas import tpu as pltpu
from jax.experimental.pallas import tpu_sc as plsc

_LANES = 16
_NWORKERS = 32
_CHUNK = 4096


def kernel(categorical_inputs, tables):
    B, T = categorical_inputs.shape
    _, V, D = tables.shape

    tab_t = jnp.transpose(tables, (0, 2, 1))
    cat_t = categorical_inputs.T

    n_pairs = T * D // _NWORKERS

    mesh = plsc.VectorSubcoreMesh(core_axis_name="core", subcore_axis_name="subcore")

    @functools.partial(
        pl.kernel,
        out_type=jax.ShapeDtypeStruct((T, D, B), tables.dtype),
        mesh=mesh,
        compiler_params=pltpu.CompilerParams(needs_layout_passes=False),
        scratch_types=[
            pltpu.VMEM((V,), jnp.float32),
            pltpu.VMEM((B,), jnp.int32),
            pltpu.VMEM((2, _CHUNK), jnp.float32),
            pltpu.SemaphoreType.DMA,
        ],
    )
    def run(tab_hbm, cat_hbm, out_hbm, vrow, idxv, outb, osem):
        wid = lax.axis_index("subcore") * 2 + lax.axis_index("core")
        p0 = wid * n_pairs

        @pl.loop(0, n_pairs)
        def pair(i):
            p = p0 + i
            t = lax.shift_right_logical(p, 5)
            d = jnp.bitwise_and(p, 31)

            @pl.when(jnp.logical_or(d == 0, i == 0))
            def _():
                pltpu.sync_copy(cat_hbm.at[t, :], idxv)

            pltpu.sync_copy(tab_hbm.at[t, d, :], vrow)

            writes = []
            for c in range(B // _CHUNK):
                s = c % 2
                if c >= 2:
                    writes[c - 2].wait()

                @pl.loop(0, _CHUNK // _LANES, step=8)
                def gath(j):
                    ivs = [
                        idxv[pl.ds(c * _CHUNK + (j + u) * _LANES, _LANES)]
                        for u in range(8)
                    ]
                    gs = [plsc.load_gather(vrow, [iv]) for iv in ivs]
                    for u in range(8):
                        outb[s, pl.ds((j + u) * _LANES, _LANES)] = gs[u]

                writes.append(
                    pltpu.async_copy(
                        outb.at[s], out_hbm.at[t, d, pl.ds(c * _CHUNK, _CHUNK)], osem
                    )
                )
            writes[-2].wait()
            writes[-1].wait()

    out_t = run(tab_t, cat_t)
    return jnp.transpose(out_t, (2, 0, 1))
